# baseline (speedup 1.0000x reference)
import functools

import jax
import jax.numpy as jnp
from jax import lax
from jax.experimental import pallas as pl
from jax.experimental.pallas import tpu as pltpu
from jax.experimental.pallas import tpu_sc as plsc

B = 16384
EMB = 128
NC = 2
NS = 16
NW = NC * NS
CHUNK = 128

SLICES = (16384,)


def _make_sc_gather(bs):
    b_per_w = bs // NW
    n_chunks = b_per_w // CHUNK
    n_total = 2 * n_chunks
    nbuf = min(n_total, 7)

    def body(user_table, item_table, users, items, u_out, i_out,
             idx_u, idx_i, bufs, sem_g, sem_s, sem_i):
        c = lax.axis_index("c")
        s = lax.axis_index("s")
        wid = s * NC + c
        base = wid * b_per_w

        cp_u = pltpu.async_copy(users.at[pl.ds(base, b_per_w)], idx_u,
                                sem_i.at[0])
        cp_i = pltpu.async_copy(items.at[pl.ds(base, b_per_w)], idx_i,
                                sem_i.at[1])
        cp_u.wait()
        cp_i.wait()

        chunks = [cfg for ch in range(n_chunks)
                  for cfg in ((user_table, idx_u, ch, u_out),
                              (item_table, idx_i, ch, i_out))]

        def fire_gather(cn, slot):
            table, idx, ch, _ = chunks[cn]
            return pltpu.async_copy(
                table.at[idx.at[pl.ds(ch * CHUNK, CHUNK)]],
                bufs.at[slot], sem_g.at[slot])

        gathers = [fire_gather(cn, cn % nbuf) for cn in range(min(nbuf, n_total))]
        gathers += [None] * (n_total - len(gathers))
        stores = [None] * n_total

        for cn in range(n_total):
            slot = cn % nbuf
            _, _, ch, out_hbm = chunks[cn]
            off = base + ch * CHUNK
            gathers[cn].wait()
            stores[cn] = pltpu.async_copy(
                bufs.at[slot], out_hbm.at[pl.ds(off, CHUNK)], sem_s.at[slot])
            nxt = cn + nbuf
            if nxt < n_total:
                stores[cn].wait()
                stores[cn] = None
                gathers[nxt] = fire_gather(nxt, slot)

        for cn in range(n_total):
            if stores[cn] is not None:
                stores[cn].wait()

    return pl.kernel(
        body,
        out_type=(
            jax.ShapeDtypeStruct((bs, EMB), jnp.float32),
            jax.ShapeDtypeStruct((bs, EMB), jnp.float32),
        ),
        mesh=plsc.VectorSubcoreMesh(core_axis_name="c", subcore_axis_name="s"),
        scratch_types=[
            pltpu.VMEM((b_per_w,), jnp.int32),
            pltpu.VMEM((b_per_w,), jnp.int32),
            pltpu.VMEM((nbuf, CHUNK, EMB), jnp.float32),
            pltpu.SemaphoreType.DMA((nbuf,)),
            pltpu.SemaphoreType.DMA((nbuf,)),
            pltpu.SemaphoreType.DMA((2,)),
        ],
    )


def _mlp_body(u_ref, i_ref, w1a_ref, w1b_ref, b1_ref, w2_ref, b2_ref, o_ref):
    u = u_ref[...].astype(jnp.bfloat16)
    it = i_ref[...].astype(jnp.bfloat16)
    dn = (((1,), (1,)), ((), ()))
    h = lax.dot_general(w1a_ref[...], u, dn, preferred_element_type=jnp.float32)
    h += lax.dot_general(w1b_ref[...], it, dn, preferred_element_type=jnp.float32)
    h = jnp.maximum(h + b1_ref[...], 0.0)
    s = lax.dot_general(w2_ref[...], h, (((1,), (0,)), ((), ())),
                        preferred_element_type=jnp.float32)
    o_ref[...] = s + b2_ref[0, 0]


def _mlp(u_emb, i_emb, w1a, w1b, b1, w2, b2):
    bs = u_emb.shape[0]
    bm = min(bs, 8192)
    grid = (bs // bm,)
    return pl.pallas_call(
        _mlp_body,
        grid=grid,
        in_specs=[
            pl.BlockSpec((bm, EMB), lambda i: (i, 0)),
            pl.BlockSpec((bm, EMB), lambda i: (i, 0)),
            pl.BlockSpec((EMB, EMB), lambda i: (0, 0)),
            pl.BlockSpec((EMB, EMB), lambda i: (0, 0)),
            pl.BlockSpec((EMB, 1), lambda i: (0, 0)),
            pl.BlockSpec((1, EMB), lambda i: (0, 0)),
            pl.BlockSpec((1, 1), lambda i: (0, 0)),
        ],
        out_specs=pl.BlockSpec((1, bm), lambda i: (0, i)),
        out_shape=jax.ShapeDtypeStruct((1, bs), jnp.float32),
    )(u_emb, i_emb, w1a, w1b, b1, w2, b2)


_sc_gathers = [_make_sc_gather(bs) for bs in SLICES]


def kernel(users, items, user_table, item_table, W1, b1, W2, b2):
    w1a = W1[:, :EMB].astype(jnp.bfloat16)
    w1b = W1[:, EMB:].astype(jnp.bfloat16)
    b1r = b1.reshape(EMB, 1)
    w2r = W2.reshape(1, EMB)
    b2r = b2.reshape(1, 1)

    gathered = []
    off = 0
    for g, bs in zip(_sc_gathers, SLICES):
        gathered.append(g(user_table, item_table,
                          lax.slice(users, (off,), (off + bs,)),
                          lax.slice(items, (off,), (off + bs,))))
        off += bs
    scores = [_mlp(u, i, w1a, w1b, b1r, w2r, b2r) for u, i in gathered]
    return jnp.concatenate(scores, axis=1).reshape(B)


# speedup vs baseline: 1.0028x; 1.0028x over previous
import functools

import jax
import jax.numpy as jnp
from jax import lax
from jax.experimental import pallas as pl
from jax.experimental.pallas import tpu as pltpu
from jax.experimental.pallas import tpu_sc as plsc

B = 16384
EMB = 128
NC = 2
NS = 16
NW = NC * NS
CHUNK = 128

SLICES = (16384,)


def _make_sc_gather(bs):
    b_per_w = bs // NW
    n_chunks = b_per_w // CHUNK
    n_total = 2 * n_chunks
    nbuf = min(n_total, 7)

    def body(user_table, item_table, users, items, u_out, i_out,
             idx_u, idx_i, bufs, sem_g, sem_s, sem_i):
        c = lax.axis_index("c")
        s = lax.axis_index("s")
        wid = s * NC + c
        base = wid * b_per_w

        cp_u = pltpu.async_copy(users.at[pl.ds(base, b_per_w)], idx_u,
                                sem_i.at[0])
        cp_i = pltpu.async_copy(items.at[pl.ds(base, b_per_w)], idx_i,
                                sem_i.at[1])
        cp_u.wait()
        cp_i.wait()

        chunks = [(user_table, idx_u, ch, u_out) for ch in range(n_chunks)] + \
                 [(item_table, idx_i, ch, i_out) for ch in range(n_chunks)]

        def fire_gather(cn, slot):
            table, idx, ch, _ = chunks[cn]
            return pltpu.async_copy(
                table.at[idx.at[pl.ds(ch * CHUNK, CHUNK)]],
                bufs.at[slot], sem_g.at[slot])

        gathers = [fire_gather(cn, cn % nbuf) for cn in range(min(nbuf, n_total))]
        gathers += [None] * (n_total - len(gathers))
        stores = [None] * n_total

        for cn in range(n_total):
            slot = cn % nbuf
            _, _, ch, out_hbm = chunks[cn]
            off = base + ch * CHUNK
            gathers[cn].wait()
            stores[cn] = pltpu.async_copy(
                bufs.at[slot], out_hbm.at[pl.ds(off, CHUNK)], sem_s.at[slot])
            nxt = cn + nbuf
            if nxt < n_total:
                stores[cn].wait()
                stores[cn] = None
                gathers[nxt] = fire_gather(nxt, slot)

        for cn in range(n_total):
            if stores[cn] is not None:
                stores[cn].wait()

    return pl.kernel(
        body,
        out_type=(
            jax.ShapeDtypeStruct((bs, EMB), jnp.float32),
            jax.ShapeDtypeStruct((bs, EMB), jnp.float32),
        ),
        mesh=plsc.VectorSubcoreMesh(core_axis_name="c", subcore_axis_name="s"),
        scratch_types=[
            pltpu.VMEM((b_per_w,), jnp.int32),
            pltpu.VMEM((b_per_w,), jnp.int32),
            pltpu.VMEM((nbuf, CHUNK, EMB), jnp.float32),
            pltpu.SemaphoreType.DMA((nbuf,)),
            pltpu.SemaphoreType.DMA((nbuf,)),
            pltpu.SemaphoreType.DMA((2,)),
        ],
    )


def _mlp_body(u_ref, i_ref, w1a_ref, w1b_ref, b1_ref, w2_ref, b2_ref, o_ref):
    u = u_ref[...].astype(jnp.bfloat16)
    it = i_ref[...].astype(jnp.bfloat16)
    dn = (((1,), (1,)), ((), ()))
    h = lax.dot_general(w1a_ref[...], u, dn, preferred_element_type=jnp.float32)
    h += lax.dot_general(w1b_ref[...], it, dn, preferred_element_type=jnp.float32)
    h = jnp.maximum(h + b1_ref[...], 0.0)
    s = lax.dot_general(w2_ref[...], h, (((1,), (0,)), ((), ())),
                        preferred_element_type=jnp.float32)
    o_ref[...] = s + b2_ref[0, 0]


def _mlp(u_emb, i_emb, w1a, w1b, b1, w2, b2):
    bs = u_emb.shape[0]
    bm = min(bs, 8192)
    grid = (bs // bm,)
    return pl.pallas_call(
        _mlp_body,
        grid=grid,
        in_specs=[
            pl.BlockSpec((bm, EMB), lambda i: (i, 0)),
            pl.BlockSpec((bm, EMB), lambda i: (i, 0)),
            pl.BlockSpec((EMB, EMB), lambda i: (0, 0)),
            pl.BlockSpec((EMB, EMB), lambda i: (0, 0)),
            pl.BlockSpec((EMB, 1), lambda i: (0, 0)),
            pl.BlockSpec((1, EMB), lambda i: (0, 0)),
            pl.BlockSpec((1, 1), lambda i: (0, 0)),
        ],
        out_specs=pl.BlockSpec((1, bm), lambda i: (0, i)),
        out_shape=jax.ShapeDtypeStruct((1, bs), jnp.float32),
    )(u_emb, i_emb, w1a, w1b, b1, w2, b2)


_sc_gathers = [_make_sc_gather(bs) for bs in SLICES]


def kernel(users, items, user_table, item_table, W1, b1, W2, b2):
    w1a = W1[:, :EMB].astype(jnp.bfloat16)
    w1b = W1[:, EMB:].astype(jnp.bfloat16)
    b1r = b1.reshape(EMB, 1)
    w2r = W2.reshape(1, EMB)
    b2r = b2.reshape(1, 1)

    gathered = []
    off = 0
    for g, bs in zip(_sc_gathers, SLICES):
        gathered.append(g(user_table, item_table,
                          lax.slice(users, (off,), (off + bs,)),
                          lax.slice(items, (off,), (off + bs,))))
        off += bs
    scores = [_mlp(u, i, w1a, w1b, b1r, w2r, b2r) for u, i in gathered]
    return jnp.concatenate(scores, axis=1).reshape(B)
